# baseline (speedup 1.0000x reference)
import functools

import jax
import jax.numpy as jnp
from jax import lax
from jax.experimental import pallas as pl
from jax.experimental.pallas import tpu as pltpu
from jax.experimental.pallas import tpu_sc as plsc

N = 8192
K = 8192
D = 32
TBLK = 512
GRID = N // TBLK

_NC, _NS = 2, 16
NW = _NC * _NS
BPW = N // NW


def _dist_argmin_body(z_ref, e_ref, idx_ref, dsum_ref, epad_ref):
    i = pl.program_id(0)
    z_blk = z_ref[...]
    e = e_ref[...]
    z2 = jnp.sum(z_blk * z_blk, axis=1, keepdims=True)
    e2 = jnp.sum(e * e, axis=1)
    mm2 = lax.dot_general((z_blk * 2.0).astype(jnp.bfloat16),
                          e.astype(jnp.bfloat16),
                          (((1,), (1,)), ((), ())),
                          preferred_element_type=jnp.float32)
    d = (z2 + e2[None, :]) - mm2
    half = K // 2
    d0 = d[:, :half]
    d1 = d[:, half:]
    m0 = jnp.min(d0, axis=1, keepdims=True)
    m1 = jnp.min(d1, axis=1, keepdims=True)
    iota = lax.broadcasted_iota(jnp.int32, d0.shape, 1).astype(jnp.float32)
    big = jnp.float32(K)
    i0 = jnp.min(jnp.where(d0 == m0, iota, big), axis=1).astype(jnp.int32)
    i1 = jnp.min(jnp.where(d1 == m1, iota, big), axis=1).astype(jnp.int32) + half
    m0b = m0.astype(jnp.bfloat16).astype(jnp.float32)
    win1 = m1 < m0b
    idx = jnp.where(win1[:, 0], i1, i0)
    dmin = jnp.where(win1, m1, m0)
    idx_ref[0, 0, :] = idx

    @pl.when(i == 0)
    def _zero():
        dsum_ref[...] = jnp.zeros_like(dsum_ref)
        epad_ref[...] = jnp.zeros_like(epad_ref)
        epad_ref[:, :D] = e.astype(jnp.bfloat16).astype(jnp.float32)

    dsum_ref[...] += jnp.sum(dmin, axis=0, keepdims=True)


_dist_argmin = pl.pallas_call(
    _dist_argmin_body,
    grid=(GRID,),
    in_specs=[
        pl.BlockSpec((TBLK, D), lambda i: (i, 0)),
        pl.BlockSpec((K, D), lambda i: (0, 0)),
    ],
    out_specs=[
        pl.BlockSpec((1, 1, TBLK), lambda i: (i, 0, 0)),
        pl.BlockSpec((1, 1), lambda i: (0, 0)),
        pl.BlockSpec((K, 128), lambda i: (0, 0)),
    ],
    out_shape=[
        jax.ShapeDtypeStruct((GRID, 1, TBLK), jnp.int32),
        jax.ShapeDtypeStruct((1, 1), jnp.float32),
        jax.ShapeDtypeStruct((K, 128), jnp.float32),
    ],
)


DP = 128


@functools.partial(
    pl.kernel,
    mesh=plsc.VectorSubcoreMesh(core_axis_name="c", subcore_axis_name="s"),
    out_type=jax.ShapeDtypeStruct((N, DP), jnp.float32),
    scratch_types=[
        pltpu.VMEM((BPW,), jnp.int32),
        pltpu.VMEM((BPW, DP), jnp.float32),
        pltpu.SemaphoreType.DMA,
    ],
)
def _sc_gather(table_hbm, idx_hbm, out_hbm, idx_v, rows_v, sem):
    wid = lax.axis_index("s") * _NC + lax.axis_index("c")
    base = wid * BPW
    pltpu.sync_copy(idx_hbm.at[pl.ds(base, BPW)], idx_v)
    pltpu.async_copy(table_hbm.at[idx_v], rows_v, sem).wait()
    pltpu.sync_copy(rows_v, out_hbm.at[pl.ds(base, BPW)])


def kernel(z, codebook_weight):
    e = codebook_weight
    zf = z.reshape(N, D)
    idx3, dsum, e_pad = _dist_argmin(zf, e)
    idx = idx3.reshape(N)
    z_q = _sc_gather(e_pad, idx)[:, :D]
    loss = dsum[0, 0] / jnp.float32(N * D)
    return loss, z_q.reshape(z.shape), idx.reshape(z.shape[0], z.shape[1])


# speedup vs baseline: 1.0908x; 1.0908x over previous
import functools

import jax
import jax.numpy as jnp
from jax import lax
from jax.experimental import pallas as pl
from jax.experimental.pallas import tpu as pltpu
from jax.experimental.pallas import tpu_sc as plsc

N = 8192
K = 8192
D = 32
TBLK = 1024
GRID = N // TBLK

_NC, _NS = 2, 16
NW = _NC * _NS
BPW = N // NW


def _dist_argmin_body(z_ref, e_ref, idx_ref, dsum_ref, epad_ref):
    i = pl.program_id(0)
    z_blk = z_ref[...]
    e = e_ref[...]
    z2 = jnp.sum(z_blk * z_blk, axis=1, keepdims=True)
    e2 = jnp.sum(e * e, axis=1)
    mm = lax.dot_general(z_blk.astype(jnp.bfloat16), e.astype(jnp.bfloat16),
                         (((1,), (1,)), ((), ())),
                         preferred_element_type=jnp.float32)
    d = (z2 + e2[None, :]) - 2.0 * mm
    half = K // 2
    d0 = d[:, :half]
    d1 = d[:, half:]
    m0 = jnp.min(d0, axis=1, keepdims=True)
    m1 = jnp.min(d1, axis=1, keepdims=True)
    iota = lax.broadcasted_iota(jnp.int32, d0.shape, 1).astype(jnp.float32)
    big = jnp.float32(K)
    i0 = jnp.min(jnp.where(d0 == m0, iota, big), axis=1).astype(jnp.int32)
    i1 = jnp.min(jnp.where(d1 == m1, iota, big), axis=1).astype(jnp.int32) + half
    m0b = m0.astype(jnp.bfloat16).astype(jnp.float32)
    win1 = m1 < m0b
    idx = jnp.where(win1[:, 0], i1, i0)
    dmin = jnp.where(win1, m1, m0)
    idx_ref[0, 0, :] = idx

    @pl.when(i == 0)
    def _zero():
        dsum_ref[...] = jnp.zeros_like(dsum_ref)
        epad_ref[...] = jnp.zeros_like(epad_ref)
        epad_ref[:, :D] = e.astype(jnp.bfloat16).astype(jnp.float32)

    dsum_ref[...] += jnp.sum(dmin, axis=0, keepdims=True)


_dist_argmin = pl.pallas_call(
    _dist_argmin_body,
    grid=(GRID,),
    in_specs=[
        pl.BlockSpec((TBLK, D), lambda i: (i, 0)),
        pl.BlockSpec((K, D), lambda i: (0, 0)),
    ],
    out_specs=[
        pl.BlockSpec((1, 1, TBLK), lambda i: (i, 0, 0)),
        pl.BlockSpec((1, 1), lambda i: (0, 0)),
        pl.BlockSpec((K, 128), lambda i: (0, 0)),
    ],
    out_shape=[
        jax.ShapeDtypeStruct((GRID, 1, TBLK), jnp.int32),
        jax.ShapeDtypeStruct((1, 1), jnp.float32),
        jax.ShapeDtypeStruct((K, 128), jnp.float32),
    ],
)


DP = 128


@functools.partial(
    pl.kernel,
    mesh=plsc.VectorSubcoreMesh(core_axis_name="c", subcore_axis_name="s"),
    out_type=jax.ShapeDtypeStruct((N, DP), jnp.float32),
    scratch_types=[
        pltpu.VMEM((BPW,), jnp.int32),
        pltpu.VMEM((BPW, DP), jnp.float32),
        pltpu.SemaphoreType.DMA,
    ],
)
def _sc_gather(table_hbm, idx_hbm, out_hbm, idx_v, rows_v, sem):
    wid = lax.axis_index("s") * _NC + lax.axis_index("c")
    base = wid * BPW
    pltpu.sync_copy(idx_hbm.at[pl.ds(base, BPW)], idx_v)
    pltpu.async_copy(table_hbm.at[idx_v], rows_v, sem).wait()
    pltpu.sync_copy(rows_v, out_hbm.at[pl.ds(base, BPW)])


def kernel(z, codebook_weight):
    e = codebook_weight
    zf = z.reshape(N, D)
    idx3, dsum, e_pad = _dist_argmin(zf, e)
    idx = idx3.reshape(N)
    z_q = _sc_gather(e_pad, idx)[:, :D]
    loss = dsum[0, 0] / jnp.float32(N * D)
    return loss, z_q.reshape(z.shape), idx.reshape(z.shape[0], z.shape[1])
